# baseline (speedup 1.0000x reference)
import jax
import jax.numpy as jnp
from jax.experimental import pallas as pl
from jax.experimental.pallas import tpu as pltpu

_B, _C, _H, _W = 8, 256, 32, 32
_K = 1024
_P = _H * _W
_BETA = 0.25
_M = _B * _P * _C


def _vq_kernel(x_ref, kw_ref, x2_ref, k2_ref, vw_ref,
               ste_ref, grad_ref, idx_ref, loss_ref, pcbt_ref, acc_ref):
    b = pl.program_id(0)
    xb = x_ref[0].reshape(_C, _P)
    kw = kw_ref[...]

    @pl.when(b == 0)
    def _init():
        pcbt_ref[...] = jax.lax.dot_general(
            vw_ref[...], kw, (((0,), (1,)), ((), ())),
            preferred_element_type=jnp.float32)
        acc_ref[0] = 0.0

    ab = jax.lax.dot_general(xb.T, kw, (((1,), (1,)), ((), ())),
                             preferred_element_type=jnp.float32)
    d = (x2_ref[0] + k2_ref[...]) - 2.0 * ab
    dmin = jnp.min(d, axis=1, keepdims=True)
    cols = jax.lax.broadcasted_iota(jnp.int32, (_P, _K), 1)
    idx = jnp.min(jnp.where(d == dmin, cols, _K), axis=1).astype(jnp.int32)
    idx_ref[0, 0, :] = idx

    onehot_t = (jax.lax.broadcasted_iota(jnp.int32, (_K, _P), 0)
                == idx[None, :]).astype(jnp.float32)
    xq = jax.lax.dot_general(pcbt_ref[...], onehot_t, (((1,), (0,)), ((), ())),
                             preferred_element_type=jnp.float32)

    diff = xq - xb
    ste_ref[...] = (xb + diff).reshape(1, _C, _H, _W)
    grad_ref[...] = xq.reshape(1, _C, _H, _W)
    acc_ref[0] += jnp.sum(diff * diff)

    @pl.when(b == _B - 1)
    def _fin():
        m = acc_ref[0] / _M
        loss_ref[0, 0] = m + _BETA * m


def kernel(x, key_weight, value_weight):
    xt = jnp.transpose(x, (0, 2, 3, 1))
    xf = xt.reshape(-1, xt.shape[-1])
    x2 = jnp.sum(xf ** 2, axis=1, keepdims=True).reshape(_B, _P, 1)
    k2 = jnp.sum(key_weight ** 2, axis=1).reshape(1, _K)

    out_shapes = [
        jax.ShapeDtypeStruct((_B, _C, _H, _W), jnp.float32),
        jax.ShapeDtypeStruct((_B, _C, _H, _W), jnp.float32),
        jax.ShapeDtypeStruct((_B, 1, _P), jnp.int32),
        jax.ShapeDtypeStruct((1, 1), jnp.float32),
    ]
    ste, grad, idx, loss = pl.pallas_call(
        _vq_kernel,
        grid=(_B,),
        in_specs=[
            pl.BlockSpec((1, _C, _H, _W), lambda b: (b, 0, 0, 0)),
            pl.BlockSpec((_K, _C), lambda b: (0, 0)),
            pl.BlockSpec((1, _P, 1), lambda b: (b, 0, 0)),
            pl.BlockSpec((1, _K), lambda b: (0, 0)),
            pl.BlockSpec((_C, _C), lambda b: (0, 0)),
        ],
        out_specs=[
            pl.BlockSpec((1, _C, _H, _W), lambda b: (b, 0, 0, 0)),
            pl.BlockSpec((1, _C, _H, _W), lambda b: (b, 0, 0, 0)),
            pl.BlockSpec((1, 1, _P), lambda b: (b, 0, 0)),
            pl.BlockSpec(memory_space=pltpu.SMEM),
        ],
        out_shape=out_shapes,
        scratch_shapes=[
            pltpu.VMEM((_C, _K), jnp.float32),
            pltpu.SMEM((1,), jnp.float32),
        ],
        compiler_params=pltpu.CompilerParams(
            dimension_semantics=("arbitrary",)),
    )(x, key_weight, x2, k2, value_weight)

    return (ste, grad, idx.reshape(_B * _P), loss[0, 0])


# speedup vs baseline: 2.3164x; 2.3164x over previous
import jax
import jax.numpy as jnp
from jax.experimental import pallas as pl
from jax.experimental.pallas import tpu as pltpu

_B, _C, _H, _W = 8, 256, 32, 32
_K = 1024
_P = _H * _W
_BETA = 0.25
_M = _B * _P * _C


def _vq_kernel(x_ref, kw_ref, x2_ref, k2_ref, vw_ref,
               ste_ref, grad_ref, idx_ref, loss_ref, pcbt_ref, acc_ref):
    b = pl.program_id(0)
    xb = x_ref[0]
    kw = kw_ref[...]

    @pl.when(b == 0)
    def _init():
        pcbt_ref[...] = jax.lax.dot_general(
            vw_ref[...], kw, (((0,), (1,)), ((), ())),
            preferred_element_type=jnp.float32)
        acc_ref[0] = 0.0

    ab = jax.lax.dot_general(xb.T, kw, (((1,), (1,)), ((), ())),
                             preferred_element_type=jnp.float32)
    d = (x2_ref[0].T + k2_ref[...]) - 2.0 * ab
    dmin = jnp.min(d, axis=1, keepdims=True)
    cols = jax.lax.broadcasted_iota(jnp.int32, (_P, _K), 1)
    idx = jnp.min(jnp.where(d == dmin, cols, _K), axis=1).astype(jnp.int32)
    idx_ref[0, 0, :] = idx

    onehot_t = (jax.lax.broadcasted_iota(jnp.int32, (_K, _P), 0)
                == idx[None, :]).astype(jnp.float32)
    xq = jax.lax.dot_general(pcbt_ref[...], onehot_t, (((1,), (0,)), ((), ())),
                             preferred_element_type=jnp.float32)

    diff = xq - xb
    ste_ref[0] = xb + diff
    grad_ref[0] = xq
    acc_ref[0] += jnp.sum(diff * diff)

    @pl.when(b == _B - 1)
    def _fin():
        m = acc_ref[0] / _M
        loss_ref[0, 0] = m + _BETA * m


def kernel(x, key_weight, value_weight):
    xt = jnp.transpose(x, (0, 2, 3, 1))
    xf = xt.reshape(-1, xt.shape[-1])
    x2 = jnp.sum(xf ** 2, axis=1, keepdims=True).reshape(_B, 1, _P)
    k2 = jnp.sum(key_weight ** 2, axis=1).reshape(1, _K)
    xr = x.reshape(_B, _C, _P)

    out_shapes = [
        jax.ShapeDtypeStruct((_B, _C, _P), jnp.float32),
        jax.ShapeDtypeStruct((_B, _C, _P), jnp.float32),
        jax.ShapeDtypeStruct((_B, 1, _P), jnp.int32),
        jax.ShapeDtypeStruct((1, 1), jnp.float32),
    ]
    ste, grad, idx, loss = pl.pallas_call(
        _vq_kernel,
        grid=(_B,),
        in_specs=[
            pl.BlockSpec((1, _C, _P), lambda b: (b, 0, 0)),
            pl.BlockSpec((_K, _C), lambda b: (0, 0)),
            pl.BlockSpec((1, 1, _P), lambda b: (b, 0, 0)),
            pl.BlockSpec((1, _K), lambda b: (0, 0)),
            pl.BlockSpec((_C, _C), lambda b: (0, 0)),
        ],
        out_specs=[
            pl.BlockSpec((1, _C, _P), lambda b: (b, 0, 0)),
            pl.BlockSpec((1, _C, _P), lambda b: (b, 0, 0)),
            pl.BlockSpec((1, 1, _P), lambda b: (b, 0, 0)),
            pl.BlockSpec(memory_space=pltpu.SMEM),
        ],
        out_shape=out_shapes,
        scratch_shapes=[
            pltpu.VMEM((_C, _K), jnp.float32),
            pltpu.SMEM((1,), jnp.float32),
        ],
        compiler_params=pltpu.CompilerParams(
            dimension_semantics=("arbitrary",)),
    )(xr, key_weight, x2, k2, value_weight)

    return (ste.reshape(_B, _C, _H, _W), grad.reshape(_B, _C, _H, _W),
            idx.reshape(_B * _P), loss[0, 0])


# speedup vs baseline: 2.6219x; 1.1319x over previous
import jax
import jax.numpy as jnp
from jax.experimental import pallas as pl
from jax.experimental.pallas import tpu as pltpu

_B, _C, _H, _W = 8, 256, 32, 32
_K = 1024
_P = _H * _W
_BETA = 0.25
_M = _B * _P * _C


def _vq_kernel(x_ref, kw_ref, vw_ref,
               ste_ref, grad_ref, idx_ref, loss_ref,
               pcbt_ref, k2_ref, acc_ref):
    b = pl.program_id(0)
    xb = x_ref[0]
    kw = kw_ref[...]

    @pl.when(b == 0)
    def _init():
        pcbt_ref[...] = jax.lax.dot_general(
            vw_ref[...], kw, (((0,), (1,)), ((), ())),
            preferred_element_type=jnp.float32)
        k2_ref[...] = jnp.sum(kw * kw, axis=1)[None, :]
        acc_ref[0] = 0.0

    xbt = xb.T
    ab = jax.lax.dot_general(xbt, kw, (((1,), (1,)), ((), ())),
                             preferred_element_type=jnp.float32)
    x2 = jnp.sum(xbt * xbt, axis=1, keepdims=True)
    d = (x2 + k2_ref[...]) - 2.0 * ab
    dmin = jnp.min(d, axis=1, keepdims=True)
    cols = jax.lax.broadcasted_iota(jnp.int32, (_P, _K), 1)
    idx = jnp.min(jnp.where(d == dmin, cols, _K), axis=1).astype(jnp.int32)
    idx_ref[0, 0, :] = idx

    onehot_t = (jax.lax.broadcasted_iota(jnp.int32, (_K, _P), 0)
                == idx[None, :]).astype(jnp.float32)
    xq = jax.lax.dot_general(pcbt_ref[...], onehot_t, (((1,), (0,)), ((), ())),
                             preferred_element_type=jnp.float32)

    diff = xq - xb
    ste_ref[0] = xb + diff
    grad_ref[0] = xq
    acc_ref[0] += jnp.sum(diff * diff)

    @pl.when(b == _B - 1)
    def _fin():
        m = acc_ref[0] / _M
        loss_ref[0, 0] = m + _BETA * m


def kernel(x, key_weight, value_weight):
    xr = x.reshape(_B, _C, _P)

    out_shapes = [
        jax.ShapeDtypeStruct((_B, _C, _P), jnp.float32),
        jax.ShapeDtypeStruct((_B, _C, _P), jnp.float32),
        jax.ShapeDtypeStruct((_B, 1, _P), jnp.int32),
        jax.ShapeDtypeStruct((1, 1), jnp.float32),
    ]
    ste, grad, idx, loss = pl.pallas_call(
        _vq_kernel,
        grid=(_B,),
        in_specs=[
            pl.BlockSpec((1, _C, _P), lambda b: (b, 0, 0)),
            pl.BlockSpec((_K, _C), lambda b: (0, 0)),
            pl.BlockSpec((_C, _C), lambda b: (0, 0)),
        ],
        out_specs=[
            pl.BlockSpec((1, _C, _P), lambda b: (b, 0, 0)),
            pl.BlockSpec((1, _C, _P), lambda b: (b, 0, 0)),
            pl.BlockSpec((1, 1, _P), lambda b: (b, 0, 0)),
            pl.BlockSpec(memory_space=pltpu.SMEM),
        ],
        out_shape=out_shapes,
        scratch_shapes=[
            pltpu.VMEM((_C, _K), jnp.float32),
            pltpu.VMEM((1, _K), jnp.float32),
            pltpu.SMEM((1,), jnp.float32),
        ],
        compiler_params=pltpu.CompilerParams(
            dimension_semantics=("arbitrary",)),
    )(xr, key_weight, value_weight)

    return (ste.reshape(_B, _C, _H, _W), grad.reshape(_B, _C, _H, _W),
            idx.reshape(_B * _P), loss[0, 0])


# speedup vs baseline: 2.6644x; 1.0162x over previous
import jax
import jax.numpy as jnp
from jax.experimental import pallas as pl
from jax.experimental.pallas import tpu as pltpu

_B, _C, _H, _W = 8, 256, 32, 32
_K = 1024
_P = _H * _W
_BETA = 0.25
_M = _B * _P * _C


def _vq_kernel(x_ref, kw_ref, vw_ref,
               ste_ref, grad_ref, idx_ref, loss_ref,
               pcbt_ref, kw2_ref, k2_ref, acc_ref):
    b = pl.program_id(0)
    xb = x_ref[0]
    kw = kw_ref[...]

    @pl.when(b == 0)
    def _init():
        pcbt_ref[...] = jax.lax.dot_general(
            vw_ref[...], kw, (((0,), (1,)), ((), ())),
            preferred_element_type=jnp.float32).astype(jnp.bfloat16)
        kw2_ref[...] = kw + kw
        k2_ref[...] = jnp.sum(kw * kw, axis=1)[:, None]
        acc_ref[0] = 0.0

    xbt = xb.T
    ab2 = jax.lax.dot_general(kw2_ref[...], xb, (((1,), (0,)), ((), ())),
                              preferred_element_type=jnp.float32)
    x2 = jnp.sum(xbt * xbt, axis=1, keepdims=True).T
    d = (x2 + k2_ref[...]) - ab2
    dmin = jnp.min(d, axis=0, keepdims=True)
    rows = jax.lax.broadcasted_iota(jnp.int32, (_K, _P), 0)
    idx = jnp.min(jnp.where(d == dmin, rows, _K), axis=0).astype(jnp.int32)
    idx_ref[0, 0, :] = idx

    onehot = (rows == idx[None, :]).astype(jnp.float32).astype(jnp.bfloat16)
    xq = jax.lax.dot_general(pcbt_ref[...], onehot, (((1,), (0,)), ((), ())),
                             preferred_element_type=jnp.float32)

    diff = xq - xb
    ste_ref[0] = xb + diff
    grad_ref[0] = xq
    acc_ref[0] += jnp.sum(diff * diff)

    @pl.when(b == _B - 1)
    def _fin():
        m = acc_ref[0] / _M
        loss_ref[0, 0] = m + _BETA * m


def kernel(x, key_weight, value_weight):
    xr = x.reshape(_B, _C, _P)

    out_shapes = [
        jax.ShapeDtypeStruct((_B, _C, _P), jnp.float32),
        jax.ShapeDtypeStruct((_B, _C, _P), jnp.float32),
        jax.ShapeDtypeStruct((_B, 1, _P), jnp.int32),
        jax.ShapeDtypeStruct((1, 1), jnp.float32),
    ]
    ste, grad, idx, loss = pl.pallas_call(
        _vq_kernel,
        grid=(_B,),
        in_specs=[
            pl.BlockSpec((1, _C, _P), lambda b: (b, 0, 0)),
            pl.BlockSpec((_K, _C), lambda b: (0, 0)),
            pl.BlockSpec((_C, _C), lambda b: (0, 0)),
        ],
        out_specs=[
            pl.BlockSpec((1, _C, _P), lambda b: (b, 0, 0)),
            pl.BlockSpec((1, _C, _P), lambda b: (b, 0, 0)),
            pl.BlockSpec((1, 1, _P), lambda b: (b, 0, 0)),
            pl.BlockSpec(memory_space=pltpu.SMEM),
        ],
        out_shape=out_shapes,
        scratch_shapes=[
            pltpu.VMEM((_C, _K), jnp.bfloat16),
            pltpu.VMEM((_K, _C), jnp.float32),
            pltpu.VMEM((_K, 1), jnp.float32),
            pltpu.SMEM((1,), jnp.float32),
        ],
        compiler_params=pltpu.CompilerParams(
            dimension_semantics=("arbitrary",)),
    )(xr, key_weight, value_weight)

    return (ste.reshape(_B, _C, _H, _W), grad.reshape(_B, _C, _H, _W),
            idx.reshape(_B * _P), loss[0, 0])


# speedup vs baseline: 2.6858x; 1.0080x over previous
import jax
import jax.numpy as jnp
from jax.experimental import pallas as pl
from jax.experimental.pallas import tpu as pltpu

_B, _C, _H, _W = 8, 256, 32, 32
_K = 1024
_P = _H * _W
_BETA = 0.25
_M = _B * _P * _C


def _vq_kernel(x_ref, kw_ref, vw_ref,
               ste_ref, grad_ref, idx_ref, loss_ref,
               pcbt_ref, kw2_ref, k2_ref, acc_ref):
    b = pl.program_id(0)
    xb = x_ref[0]
    kw = kw_ref[...]

    @pl.when(b == 0)
    def _init():
        pcbt_ref[...] = jax.lax.dot_general(
            vw_ref[...], kw, (((0,), (1,)), ((), ())),
            preferred_element_type=jnp.float32).astype(jnp.bfloat16)
        kw2_ref[...] = kw + kw
        k2_ref[...] = jnp.sum(kw * kw, axis=1)[:, None]
        acc_ref[0] = 0.0

    xbt = xb.T
    ab2 = jax.lax.dot_general(kw2_ref[...], xb, (((1,), (0,)), ((), ())),
                              preferred_element_type=jnp.float32)
    x2 = jnp.sum(xbt * xbt, axis=1, keepdims=True).T
    d = (x2 + k2_ref[...]) - ab2
    dmin = jnp.min(d, axis=0, keepdims=True)
    rows = jax.lax.broadcasted_iota(jnp.int32, (_K, _P), 0)
    idx = jnp.min(jnp.where(d == dmin, rows, _K), axis=0).astype(jnp.int32)
    idx_ref[0, 0, :] = idx

    onehot = (rows == idx[None, :]).astype(jnp.float32).astype(jnp.bfloat16)
    xq = jax.lax.dot_general(pcbt_ref[...], onehot, (((1,), (0,)), ((), ())),
                             preferred_element_type=jnp.float32)

    diff = xq - xb
    ste_ref[0] = xq
    grad_ref[0] = xq
    acc_ref[0] += jnp.sum(diff * diff)

    @pl.when(b == _B - 1)
    def _fin():
        m = acc_ref[0] / _M
        loss_ref[0, 0] = m + _BETA * m


def kernel(x, key_weight, value_weight):
    xr = x.reshape(_B, _C, _P)

    out_shapes = [
        jax.ShapeDtypeStruct((_B, _C, _P), jnp.float32),
        jax.ShapeDtypeStruct((_B, _C, _P), jnp.float32),
        jax.ShapeDtypeStruct((_B, 1, _P), jnp.int32),
        jax.ShapeDtypeStruct((1, 1), jnp.float32),
    ]
    ste, grad, idx, loss = pl.pallas_call(
        _vq_kernel,
        grid=(_B,),
        in_specs=[
            pl.BlockSpec((1, _C, _P), lambda b: (b, 0, 0)),
            pl.BlockSpec((_K, _C), lambda b: (0, 0)),
            pl.BlockSpec((_C, _C), lambda b: (0, 0)),
        ],
        out_specs=[
            pl.BlockSpec((1, _C, _P), lambda b: (b, 0, 0)),
            pl.BlockSpec((1, _C, _P), lambda b: (b, 0, 0)),
            pl.BlockSpec((1, 1, _P), lambda b: (b, 0, 0)),
            pl.BlockSpec(memory_space=pltpu.SMEM),
        ],
        out_shape=out_shapes,
        scratch_shapes=[
            pltpu.VMEM((_C, _K), jnp.bfloat16),
            pltpu.VMEM((_K, _C), jnp.float32),
            pltpu.VMEM((_K, 1), jnp.float32),
            pltpu.SMEM((1,), jnp.float32),
        ],
        compiler_params=pltpu.CompilerParams(
            dimension_semantics=("arbitrary",)),
    )(xr, key_weight, value_weight)

    return (ste.reshape(_B, _C, _H, _W), grad.reshape(_B, _C, _H, _W),
            idx.reshape(_B * _P), loss[0, 0])
